# baseline (speedup 1.0000x reference)
import functools

import jax
import jax.numpy as jnp
from jax import lax
from jax.experimental import pallas as pl
from jax.experimental.pallas import tpu as pltpu

N_ROWS = 16384
N_COLS = 1000
NBINS = 11
BR = 512
NB = N_ROWS // BR


def _tc_body(x_ref, lab_ref, lob_ref, hib_ref, out_ref, acc_scr):
    i = pl.program_id(0)

    @pl.when(i == 0)
    def _init():
        acc_scr[...] = jnp.zeros_like(acc_scr)

    x = x_ref[...]
    m = jnp.max(x, axis=1, keepdims=True)
    s = jnp.sum(jnp.exp(x - m), axis=1, keepdims=True)
    conf = 1.0 / s
    col = lax.broadcasted_iota(jnp.int32, (BR, N_COLS), 1)
    cand = jnp.where(x == m, col, N_COLS)
    idx = jnp.min(cand, axis=1, keepdims=True)
    lab = lab_ref[0]
    acc = (idx == lab).astype(jnp.float32)

    lob = lob_ref[...]
    hib = hib_ref[...]
    mask = (conf > lob) & (conf <= hib)
    cnt = jnp.sum(mask.astype(jnp.float32), axis=0, keepdims=True)
    csum = jnp.sum(jnp.where(mask, conf, 0.0), axis=0, keepdims=True)
    asum = jnp.sum(jnp.where(mask, acc, 0.0), axis=0, keepdims=True)
    acc_scr[0:1, :] += cnt
    acc_scr[1:2, :] += csum
    acc_scr[2:3, :] += asum

    @pl.when(i == NB - 1)
    def _finalize():
        counts = acc_scr[0:1, :]
        csums = acc_scr[1:2, :]
        asums = acc_scr[2:3, :]
        nonempty = counts > 0.0
        safe = jnp.maximum(counts, 1.0)
        accs = jnp.where(nonempty, asums / safe, 0.0)
        confs = jnp.where(nonempty, csums / safe, 0.0)
        prop = counts / jnp.float32(N_ROWS)
        contrib = jnp.where(nonempty, jnp.abs(confs - accs) * prop, 0.0)
        ece = jnp.sum(contrib, axis=1, keepdims=True)
        out_ref[...] = jnp.zeros_like(out_ref)
        out_ref[0:1, :] = jnp.broadcast_to(ece, (1, 128))
        out_ref[1:2, :] = accs
        out_ref[2:3, :] = confs


def kernel(logits, labels):
    boundaries = jnp.linspace(0.0, 1.0, NBINS + 1)
    lob = jnp.full((1, 128), 2.0, dtype=jnp.float32).at[0, :NBINS].set(boundaries[:NBINS])
    hib = jnp.full((1, 128), 3.0, dtype=jnp.float32).at[0, :NBINS].set(boundaries[1 : NBINS + 1])
    labels_r = labels.reshape(NB, BR, 1)

    out = pl.pallas_call(
        _tc_body,
        grid=(NB,),
        in_specs=[
            pl.BlockSpec((BR, N_COLS), lambda i: (i, 0)),
            pl.BlockSpec((1, BR, 1), lambda i: (i, 0, 0)),
            pl.BlockSpec((1, 128), lambda i: (0, 0)),
            pl.BlockSpec((1, 128), lambda i: (0, 0)),
        ],
        out_specs=pl.BlockSpec((8, 128), lambda i: (0, 0)),
        out_shape=jax.ShapeDtypeStruct((8, 128), jnp.float32),
        scratch_shapes=[pltpu.VMEM((8, 128), jnp.float32)],
    )(logits, labels_r, lob, hib)

    ece = out[0, :1]
    accs = out[1, :NBINS]
    confs = out[2, :NBINS]
    return (ece, accs, confs)


# speedup vs baseline: 1.9154x; 1.9154x over previous
import functools

import jax
import jax.numpy as jnp
from jax import lax
from jax.experimental import pallas as pl
from jax.experimental.pallas import tpu as pltpu

N_ROWS = 16384
N_COLS = 1000
NBINS = 11
BC = 256
NB = N_ROWS // BC


def _tc_body(x_ref, lab_ref, lob_ref, hib_ref, out_ref, cnt_scr, cs_scr, as_scr):
    i = pl.program_id(0)

    @pl.when(i == 0)
    def _init():
        cnt_scr[...] = jnp.zeros_like(cnt_scr)
        cs_scr[...] = jnp.zeros_like(cs_scr)
        as_scr[...] = jnp.zeros_like(as_scr)

    x = x_ref[...]
    m = jnp.max(x, axis=0, keepdims=True)
    s = jnp.sum(jnp.exp(x - m), axis=0, keepdims=True)
    conf = 1.0 / s
    row = lax.broadcasted_iota(jnp.int32, (N_COLS, BC), 0)
    cand = jnp.where(x == m, row, N_COLS)
    idx = jnp.min(cand, axis=0, keepdims=True)
    lab = lab_ref[0]
    acc = (idx == lab).astype(jnp.float32)

    lob = lob_ref[...]
    hib = hib_ref[...]
    mask = (conf > lob) & (conf <= hib)
    cnt_scr[...] += mask.astype(jnp.float32)
    cs_scr[...] += jnp.where(mask, conf, 0.0)
    as_scr[...] += jnp.where(mask, acc, 0.0)

    @pl.when(i == NB - 1)
    def _finalize():
        counts = jnp.sum(cnt_scr[...], axis=1, keepdims=True)
        csums = jnp.sum(cs_scr[...], axis=1, keepdims=True)
        asums = jnp.sum(as_scr[...], axis=1, keepdims=True)
        nonempty = counts > 0.0
        safe = jnp.maximum(counts, 1.0)
        accs = jnp.where(nonempty, asums / safe, 0.0)
        confs = jnp.where(nonempty, csums / safe, 0.0)
        prop = counts / jnp.float32(N_ROWS)
        contrib = jnp.where(nonempty, jnp.abs(confs - accs) * prop, 0.0)
        ece = jnp.sum(contrib, axis=0, keepdims=True)
        out_ref[...] = jnp.zeros_like(out_ref)
        out_ref[0:1, 0:1] = ece
        out_ref[16:32, 0:1] = accs
        out_ref[32:48, 0:1] = confs


def kernel(logits, labels):
    boundaries = jnp.linspace(0.0, 1.0, NBINS + 1)
    lob = jnp.full((16, 1), 2.0, dtype=jnp.float32).at[:NBINS, 0].set(boundaries[:NBINS])
    hib = jnp.full((16, 1), 3.0, dtype=jnp.float32).at[:NBINS, 0].set(boundaries[1 : NBINS + 1])
    xt = logits.T
    labels_r = labels.reshape(NB, 1, BC)

    out = pl.pallas_call(
        _tc_body,
        grid=(NB,),
        in_specs=[
            pl.BlockSpec((N_COLS, BC), lambda i: (0, i)),
            pl.BlockSpec((1, 1, BC), lambda i: (i, 0, 0)),
            pl.BlockSpec((16, 1), lambda i: (0, 0)),
            pl.BlockSpec((16, 1), lambda i: (0, 0)),
        ],
        out_specs=pl.BlockSpec((48, 128), lambda i: (0, 0)),
        out_shape=jax.ShapeDtypeStruct((48, 128), jnp.float32),
        scratch_shapes=[
            pltpu.VMEM((16, BC), jnp.float32),
            pltpu.VMEM((16, BC), jnp.float32),
            pltpu.VMEM((16, BC), jnp.float32),
        ],
    )(xt, labels_r, lob, hib)

    ece = out[0, :1]
    accs = out[16 : 16 + NBINS, 0]
    confs = out[32 : 32 + NBINS, 0]
    return (ece, accs, confs)


# speedup vs baseline: 2.7894x; 1.4564x over previous
import functools

import jax
import jax.numpy as jnp
import numpy as np
from jax import lax
from jax.experimental import pallas as pl
from jax.experimental.pallas import tpu as pltpu

N_ROWS = 16384
N_COLS = 1000
NBINS = 11
BC = 512
NB = N_ROWS // BC


def _tc_body(x_ref, lab_ref, lob_ref, hib_ref, out_ref, cnt_scr, cs_scr, as_scr):
    i = pl.program_id(0)

    @pl.when(i == 0)
    def _init():
        cnt_scr[...] = jnp.zeros_like(cnt_scr)
        cs_scr[...] = jnp.zeros_like(cs_scr)
        as_scr[...] = jnp.zeros_like(as_scr)

    x = x_ref[...]
    m = jnp.max(x, axis=0, keepdims=True)
    s = jnp.sum(jnp.exp(x - m), axis=0, keepdims=True)
    conf = 1.0 / s
    row = lax.broadcasted_iota(jnp.int32, (N_COLS, BC), 0)
    cand = jnp.where(x == m, row, N_COLS)
    idx = jnp.min(cand, axis=0, keepdims=True)
    lab = lab_ref[0]
    acc = (idx == lab).astype(jnp.float32)

    lob = lob_ref[...]
    hib = hib_ref[...]
    mask = (conf > lob) & (conf <= hib)
    cnt_scr[...] += mask.astype(jnp.float32)
    cs_scr[...] += jnp.where(mask, conf, 0.0)
    as_scr[...] += jnp.where(mask, acc, 0.0)

    @pl.when(i == NB - 1)
    def _finalize():
        counts = jnp.sum(cnt_scr[...], axis=1, keepdims=True)
        csums = jnp.sum(cs_scr[...], axis=1, keepdims=True)
        asums = jnp.sum(as_scr[...], axis=1, keepdims=True)
        nonempty = counts > 0.0
        safe = jnp.maximum(counts, 1.0)
        accs = jnp.where(nonempty, asums / safe, 0.0)
        confs = jnp.where(nonempty, csums / safe, 0.0)
        prop = counts / jnp.float32(N_ROWS)
        contrib = jnp.where(nonempty, jnp.abs(confs - accs) * prop, 0.0)
        ece = jnp.sum(contrib, axis=0, keepdims=True)
        out_ref[...] = jnp.zeros_like(out_ref)
        out_ref[0:1, 0:1] = ece
        out_ref[16:32, 0:1] = accs
        out_ref[32:48, 0:1] = confs


def kernel(logits, labels):
    bnp = np.arange(NBINS + 1, dtype=np.float32) * (np.float32(1.0) / np.float32(NBINS))
    bnp[-1] = 1.0
    lob_np = np.full((16, 1), 2.0, dtype=np.float32)
    lob_np[:NBINS, 0] = bnp[:NBINS]
    hib_np = np.full((16, 1), 3.0, dtype=np.float32)
    hib_np[:NBINS, 0] = bnp[1 : NBINS + 1]
    lob = jnp.asarray(lob_np)
    hib = jnp.asarray(hib_np)
    xt = logits.T
    labels_r = labels.reshape(NB, 1, BC)

    out = pl.pallas_call(
        _tc_body,
        grid=(NB,),
        in_specs=[
            pl.BlockSpec((N_COLS, BC), lambda i: (0, i)),
            pl.BlockSpec((1, 1, BC), lambda i: (i, 0, 0)),
            pl.BlockSpec((16, 1), lambda i: (0, 0)),
            pl.BlockSpec((16, 1), lambda i: (0, 0)),
        ],
        out_specs=pl.BlockSpec((48, 128), lambda i: (0, 0)),
        out_shape=jax.ShapeDtypeStruct((48, 128), jnp.float32),
        scratch_shapes=[
            pltpu.VMEM((16, BC), jnp.float32),
            pltpu.VMEM((16, BC), jnp.float32),
            pltpu.VMEM((16, BC), jnp.float32),
        ],
    )(xt, labels_r, lob, hib)

    ece = out[0, :1]
    accs = out[16 : 16 + NBINS, 0]
    confs = out[32 : 32 + NBINS, 0]
    return (ece, accs, confs)


# speedup vs baseline: 3.4368x; 1.2321x over previous
import functools

import jax
import jax.numpy as jnp
import numpy as np
from jax import lax
from jax.experimental import pallas as pl
from jax.experimental.pallas import tpu as pltpu

N_ROWS = 16384
N_COLS = 1000
NBINS = 11
BC = 1024
NB = N_ROWS // BC


def _tc_body(x_ref, lab_ref, lob_ref, hib_ref, out_ref, cnt_scr, cs_scr, as_scr):
    i = pl.program_id(0)

    @pl.when(i == 0)
    def _init():
        cnt_scr[...] = jnp.zeros_like(cnt_scr)
        cs_scr[...] = jnp.zeros_like(cs_scr)
        as_scr[...] = jnp.zeros_like(as_scr)

    x = x_ref[...]
    m = jnp.max(x, axis=0, keepdims=True)
    s = jnp.sum(jnp.exp(x - m), axis=0, keepdims=True)
    conf = 1.0 / s
    row = lax.broadcasted_iota(jnp.int32, (N_COLS, BC), 0)
    cand = jnp.where(x == m, row, N_COLS)
    idx = jnp.min(cand, axis=0, keepdims=True)
    lab = lab_ref[0]
    acc = (idx == lab).astype(jnp.float32)

    lob = lob_ref[...]
    hib = hib_ref[...]
    mask = (conf > lob) & (conf <= hib)
    cnt_scr[...] += mask.astype(jnp.float32)
    cs_scr[...] += jnp.where(mask, conf, 0.0)
    as_scr[...] += jnp.where(mask, acc, 0.0)

    @pl.when(i == NB - 1)
    def _finalize():
        counts = jnp.sum(cnt_scr[...], axis=1, keepdims=True)
        csums = jnp.sum(cs_scr[...], axis=1, keepdims=True)
        asums = jnp.sum(as_scr[...], axis=1, keepdims=True)
        nonempty = counts > 0.0
        safe = jnp.maximum(counts, 1.0)
        accs = jnp.where(nonempty, asums / safe, 0.0)
        confs = jnp.where(nonempty, csums / safe, 0.0)
        prop = counts / jnp.float32(N_ROWS)
        contrib = jnp.where(nonempty, jnp.abs(confs - accs) * prop, 0.0)
        ece = jnp.sum(contrib, axis=0, keepdims=True)
        out_ref[...] = jnp.zeros_like(out_ref)
        out_ref[0:1, 0:1] = ece
        out_ref[16:32, 0:1] = accs
        out_ref[32:48, 0:1] = confs


def kernel(logits, labels):
    bnp = np.arange(NBINS + 1, dtype=np.float32) * (np.float32(1.0) / np.float32(NBINS))
    bnp[-1] = 1.0
    lob_np = np.full((16, 1), 2.0, dtype=np.float32)
    lob_np[:NBINS, 0] = bnp[:NBINS]
    hib_np = np.full((16, 1), 3.0, dtype=np.float32)
    hib_np[:NBINS, 0] = bnp[1 : NBINS + 1]
    lob = jnp.asarray(lob_np)
    hib = jnp.asarray(hib_np)
    xt = logits.T
    labels_r = labels.reshape(NB, 1, BC)

    out = pl.pallas_call(
        _tc_body,
        grid=(NB,),
        in_specs=[
            pl.BlockSpec((N_COLS, BC), lambda i: (0, i)),
            pl.BlockSpec((1, 1, BC), lambda i: (i, 0, 0)),
            pl.BlockSpec((16, 1), lambda i: (0, 0)),
            pl.BlockSpec((16, 1), lambda i: (0, 0)),
        ],
        out_specs=pl.BlockSpec((48, 128), lambda i: (0, 0)),
        out_shape=jax.ShapeDtypeStruct((48, 128), jnp.float32),
        scratch_shapes=[
            pltpu.VMEM((16, BC), jnp.float32),
            pltpu.VMEM((16, BC), jnp.float32),
            pltpu.VMEM((16, BC), jnp.float32),
        ],
    )(xt, labels_r, lob, hib)

    ece = out[0, :1]
    accs = out[16 : 16 + NBINS, 0]
    confs = out[32 : 32 + NBINS, 0]
    return (ece, accs, confs)


# speedup vs baseline: 3.5782x; 1.0411x over previous
import functools

import jax
import jax.numpy as jnp
import numpy as np
from jax import lax
from jax.experimental import pallas as pl
from jax.experimental.pallas import tpu as pltpu

N_ROWS = 16384
N_COLS = 1000
NBINS = 11
BC = 2048
NB = N_ROWS // BC


def _tc_body(x_ref, lab_ref, lob_ref, hib_ref, out_ref, cnt_scr, cs_scr, as_scr):
    i = pl.program_id(0)

    @pl.when(i == 0)
    def _init():
        cnt_scr[...] = jnp.zeros_like(cnt_scr)
        cs_scr[...] = jnp.zeros_like(cs_scr)
        as_scr[...] = jnp.zeros_like(as_scr)

    x = x_ref[...]
    m = jnp.max(x, axis=0, keepdims=True)
    s = jnp.sum(jnp.exp(x - m), axis=0, keepdims=True)
    conf = 1.0 / s
    row = lax.broadcasted_iota(jnp.int32, (N_COLS, BC), 0)
    cand = jnp.where(x == m, row, N_COLS)
    idx = jnp.min(cand, axis=0, keepdims=True)
    lab = lab_ref[0]
    acc = (idx == lab).astype(jnp.float32)

    lob = lob_ref[...]
    hib = hib_ref[...]
    mask = (conf > lob) & (conf <= hib)
    cnt_scr[...] += mask.astype(jnp.float32)
    cs_scr[...] += jnp.where(mask, conf, 0.0)
    as_scr[...] += jnp.where(mask, acc, 0.0)

    @pl.when(i == NB - 1)
    def _finalize():
        counts = jnp.sum(cnt_scr[...], axis=1, keepdims=True)
        csums = jnp.sum(cs_scr[...], axis=1, keepdims=True)
        asums = jnp.sum(as_scr[...], axis=1, keepdims=True)
        nonempty = counts > 0.0
        safe = jnp.maximum(counts, 1.0)
        accs = jnp.where(nonempty, asums / safe, 0.0)
        confs = jnp.where(nonempty, csums / safe, 0.0)
        prop = counts / jnp.float32(N_ROWS)
        contrib = jnp.where(nonempty, jnp.abs(confs - accs) * prop, 0.0)
        ece = jnp.sum(contrib, axis=0, keepdims=True)
        out_ref[...] = jnp.zeros_like(out_ref)
        out_ref[0:1, 0:1] = ece
        out_ref[16:32, 0:1] = accs
        out_ref[32:48, 0:1] = confs


def kernel(logits, labels):
    bnp = np.arange(NBINS + 1, dtype=np.float32) * (np.float32(1.0) / np.float32(NBINS))
    bnp[-1] = 1.0
    lob_np = np.full((16, 1), 2.0, dtype=np.float32)
    lob_np[:NBINS, 0] = bnp[:NBINS]
    hib_np = np.full((16, 1), 3.0, dtype=np.float32)
    hib_np[:NBINS, 0] = bnp[1 : NBINS + 1]
    lob = jnp.asarray(lob_np)
    hib = jnp.asarray(hib_np)
    xt = logits.T
    labels_r = labels.reshape(NB, 1, BC)

    out = pl.pallas_call(
        _tc_body,
        grid=(NB,),
        in_specs=[
            pl.BlockSpec((N_COLS, BC), lambda i: (0, i)),
            pl.BlockSpec((1, 1, BC), lambda i: (i, 0, 0)),
            pl.BlockSpec((16, 1), lambda i: (0, 0)),
            pl.BlockSpec((16, 1), lambda i: (0, 0)),
        ],
        out_specs=pl.BlockSpec((48, 128), lambda i: (0, 0)),
        out_shape=jax.ShapeDtypeStruct((48, 128), jnp.float32),
        scratch_shapes=[
            pltpu.VMEM((16, BC), jnp.float32),
            pltpu.VMEM((16, BC), jnp.float32),
            pltpu.VMEM((16, BC), jnp.float32),
        ],
    )(xt, labels_r, lob, hib)

    ece = out[0, :1]
    accs = out[16 : 16 + NBINS, 0]
    confs = out[32 : 32 + NBINS, 0]
    return (ece, accs, confs)
